# Initial kernel scaffold; baseline (speedup 1.0000x reference)
import functools

import jax
import jax.numpy as jnp
from jax import lax
from jax.experimental import pallas as pl
from jax.experimental.pallas import tpu as pltpu
from jax.experimental.pallas import tpu_sc as plsc

N = 10000
E = 320000
DIN = 128
D = 256
H = 128
C = 100
NCLS = 10

NC = 2
NS = 16
NW = NC * NS
EPW = E // NW
K = 125
NCH = EPW // K
RPS = N // NS
RCH = 125
NRC = RPS // RCH

_mesh = plsc.VectorSubcoreMesh(
    core_axis_name="c", subcore_axis_name="s", num_cores=NC, num_subcores=NS)


@functools.partial(
    pl.kernel,
    out_type=jax.ShapeDtypeStruct((NC, N, 16), jnp.float32),
    mesh=_mesh,
    scratch_types=[
        pltpu.VMEM_SHARED((N, 16), jnp.float32),
        pltpu.VMEM((NCH, K), jnp.int32),
        pltpu.VMEM((K, 16), jnp.float32),
        pltpu.VMEM((RCH, 16), jnp.float32),
    ],
)
def _sc_degree(dst_hbm, deg_out, shared, idx_v, ones_v, zb_v):
    c = lax.axis_index("c")
    s = lax.axis_index("s")
    wid = c * NS + s

    one16 = jnp.full((16,), 1.0, jnp.float32)
    zero16 = jnp.zeros((16,), jnp.float32)

    def fill_ones(i, _):
        ones_v[i] = one16
        return ()
    lax.fori_loop(0, K, fill_ones, (), unroll=4)

    def fill_zeros(i, _):
        zb_v[i] = zero16
        return ()
    lax.fori_loop(0, RCH, fill_zeros, (), unroll=4)

    for t in range(NRC):
        pltpu.sync_copy(zb_v, shared.at[pl.ds(s * RPS + t * RCH, RCH)])
    plsc.subcore_barrier()

    pltpu.sync_copy(dst_hbm.at[wid], idx_v)

    def body(j, _):
        pltpu.sync_copy(ones_v, shared.at[idx_v.at[j]], add=True)
        return ()
    lax.fori_loop(0, NCH, body, ())
    plsc.subcore_barrier()

    for t in range(NRC):
        base = s * RPS + t * RCH
        pltpu.sync_copy(shared.at[pl.ds(base, RCH)], zb_v)
        pltpu.sync_copy(zb_v, deg_out.at[c, pl.ds(base, RCH)])


@functools.partial(
    pl.kernel,
    out_type=jax.ShapeDtypeStruct((NC, N, H), jnp.float32),
    mesh=_mesh,
    scratch_types=[
        pltpu.VMEM_SHARED((N, H), jnp.float32),
        pltpu.VMEM((NCH, K), jnp.int32),
        pltpu.VMEM((NCH, K), jnp.int32),
        pltpu.VMEM((K, H), jnp.float32),
        pltpu.VMEM((RCH, H), jnp.float32),
        pltpu.SemaphoreType.DMA,
    ],
)
def _sc_aggregate(table_hbm, src_hbm, dst_hbm, agg_out,
                  shared, src_v, dst_v, rows_v, zb_v, sem):
    c = lax.axis_index("c")
    s = lax.axis_index("s")
    wid = c * NS + s

    zero16 = jnp.zeros((16,), jnp.float32)

    def fill_zeros(i, _):
        for q in range(H // 16):
            zb_v[i, pl.ds(q * 16, 16)] = zero16
        return ()
    lax.fori_loop(0, RCH, fill_zeros, (), unroll=2)

    for t in range(NRC):
        pltpu.sync_copy(zb_v, shared.at[pl.ds(s * RPS + t * RCH, RCH)])
    plsc.subcore_barrier()

    pltpu.sync_copy(src_hbm.at[c, wid], src_v)
    pltpu.sync_copy(dst_hbm.at[wid], dst_v)

    def body(j, _):
        pltpu.async_copy(table_hbm.at[src_v.at[j]], rows_v, sem).wait()
        pltpu.sync_copy(rows_v, shared.at[dst_v.at[j]], add=True)
        return ()
    lax.fori_loop(0, NCH, body, ())
    plsc.subcore_barrier()

    for t in range(NRC):
        base = s * RPS + t * RCH
        pltpu.sync_copy(shared.at[pl.ds(base, RCH)], zb_v)
        pltpu.sync_copy(zb_v, agg_out.at[c, pl.ds(base, RCH)])


RB = 1000
NB = N // RB


def _norm_from_deg(deg_blk):
    deg = deg_blk[0] + deg_blk[1]
    d = jnp.maximum(deg[:, 0:1], 1.0)
    return lax.rsqrt(d)


def _tc_embed_body(data_ref, we_ref, w1_ref, deg_ref, out_ref):
    x0 = jnp.dot(data_ref[...], we_ref[...], preferred_element_type=jnp.float32)
    hw = jnp.dot(x0, w1_ref[...], preferred_element_type=jnp.float32)
    hwn = hw * _norm_from_deg(deg_ref[...])
    out_ref[0] = hwn[:, :H]
    out_ref[1] = hwn[:, H:]


def _tc_layer2_body(agg_ref, deg_ref, w2_ref, out_ref):
    norm = _norm_from_deg(deg_ref[...])
    a = agg_ref[...]
    x1 = jnp.maximum(jnp.concatenate([a[0], a[1]], axis=1) * norm, 0.0)
    hw = jnp.dot(x1, w2_ref[...], preferred_element_type=jnp.float32)
    hwn = hw * norm
    out_ref[0] = hwn[:, :H]
    out_ref[1] = hwn[:, H:]


def _tc_readout_body(agg_ref, deg_ref, cent_ref, wout_ref, bout_ref,
                     out_ref, acc_ref):
    b = pl.program_id(0)

    @pl.when(b == 0)
    def _init():
        acc_ref[...] = jnp.zeros_like(acc_ref)

    norm = _norm_from_deg(deg_ref[...])
    a = agg_ref[...]
    x2 = jnp.maximum(jnp.concatenate([a[0], a[1]], axis=1) * norm, 0.0)
    cent = cent_ref[...]
    x2s = jnp.sum(x2 * x2, axis=1, keepdims=True)
    c2s = jnp.sum(cent * cent, axis=1)[None, :]
    prod = lax.dot_general(x2, cent, (((1,), (1,)), ((), ())),
                           preferred_element_type=jnp.float32)
    d2 = jnp.maximum(x2s + c2s - 2.0 * prod, 0.0)
    dist = jnp.sqrt(d2 + 1e-8)
    acc_ref[...] += jnp.sum(dist.reshape(RB // 8, 8, 128), axis=0)

    @pl.when(b == NB - 1)
    def _final():
        g = jnp.sum(acc_ref[...], axis=0, keepdims=True) * (1.0 / N)
        logits = jnp.dot(g, wout_ref[...],
                         preferred_element_type=jnp.float32)
        out_ref[...] = jnp.broadcast_to(logits, (8, 128)) + bout_ref[...]


def kernel(data, edge_index, W_embed, W_conv1, W_conv2, centroids, W_out, b_out):
    src = edge_index[0].reshape(NW, NCH, K)
    dst = edge_index[1].reshape(NW, NCH, K)
    src2 = jnp.stack([src, src + N])

    cent_pad = jnp.zeros((128, D), jnp.float32).at[:C].set(centroids)
    wout_pad = jnp.zeros((128, 128), jnp.float32).at[:C, :NCLS].set(W_out)
    bout_pad = jnp.zeros((8, 128), jnp.float32).at[:, :NCLS].set(b_out[None, :])

    deg_p = _sc_degree(dst)

    deg_spec = pl.BlockSpec((NC, RB, 16), lambda b: (0, b, 0))
    half_spec = pl.BlockSpec((NC, RB, H), lambda b: (0, b, 0))

    hwn1 = pl.pallas_call(
        _tc_embed_body,
        grid=(NB,),
        in_specs=[
            pl.BlockSpec((RB, DIN), lambda b: (b, 0)),
            pl.BlockSpec((DIN, D), lambda b: (0, 0)),
            pl.BlockSpec((D, D), lambda b: (0, 0)),
            deg_spec,
        ],
        out_specs=half_spec,
        out_shape=jax.ShapeDtypeStruct((NC, N, H), jnp.float32),
    )(data, W_embed, W_conv1, deg_p)

    agg1 = _sc_aggregate(hwn1.reshape(NC * N, H), src2, dst)

    hwn2 = pl.pallas_call(
        _tc_layer2_body,
        grid=(NB,),
        in_specs=[
            half_spec,
            deg_spec,
            pl.BlockSpec((D, D), lambda b: (0, 0)),
        ],
        out_specs=half_spec,
        out_shape=jax.ShapeDtypeStruct((NC, N, H), jnp.float32),
    )(agg1, deg_p, W_conv2)

    agg2 = _sc_aggregate(hwn2.reshape(NC * N, H), src2, dst)

    out_pad = pl.pallas_call(
        _tc_readout_body,
        grid=(NB,),
        in_specs=[
            half_spec,
            deg_spec,
            pl.BlockSpec((128, D), lambda b: (0, 0)),
            pl.BlockSpec((128, 128), lambda b: (0, 0)),
            pl.BlockSpec((8, 128), lambda b: (0, 0)),
        ],
        out_specs=pl.BlockSpec((8, 128), lambda b: (0, 0)),
        out_shape=jax.ShapeDtypeStruct((8, 128), jnp.float32),
        scratch_shapes=[pltpu.VMEM((8, 128), jnp.float32)],
    )(agg2, deg_p, cent_pad, wout_pad, bout_pad)

    return out_pad[0, :NCLS]


# speedup vs baseline: 5.4323x; 5.4323x over previous
import functools

import jax
import jax.numpy as jnp
from jax import lax
from jax.experimental import pallas as pl
from jax.experimental.pallas import tpu as pltpu
from jax.experimental.pallas import tpu_sc as plsc

N = 10000
E = 320000
DIN = 128
D = 256
H = 128
C = 100
NCLS = 10

NC = 2
NS = 16
NW = NC * NS
EPW = E // NW
K = 80
NCH = EPW // K
EPT = E // NS
NCHT = EPT // K
RPT = 640
RCH = 80

_mesh = plsc.VectorSubcoreMesh(
    core_axis_name="c", subcore_axis_name="s", num_cores=NC, num_subcores=NS)


@functools.partial(
    pl.kernel,
    out_type=jax.ShapeDtypeStruct((NC, N, H), jnp.float32),
    mesh=_mesh,
    scratch_types=[
        pltpu.VMEM_SHARED((N, H), jnp.float32),
        pltpu.VMEM((K,), jnp.int32),
        pltpu.VMEM((K, H), jnp.float32),
    ],
)
def _sc_degree(dst_hbm, deg_out, shared, idx_s, buf_v):
    c = lax.axis_index("c")
    s = lax.axis_index("s")
    wid = c * NS + s
    ncopies = jnp.where(s == NS - 1, 5, 8)

    def fill(val):
        v16 = jnp.full((16,), val, jnp.float32)

        def fbody(i, _):
            for q in range(H // 16):
                buf_v[i, pl.ds(q * 16, 16)] = v16
            return ()
        lax.fori_loop(0, K, fbody, (), unroll=2)

    fill(0.0)

    def zero_chunk(t, _):
        pltpu.sync_copy(buf_v, shared.at[pl.ds(s * RPT + t * RCH, RCH)])
        return ()
    lax.fori_loop(0, ncopies, zero_chunk, ())
    plsc.subcore_barrier()

    fill(1.0)

    def body(j, _):
        pltpu.sync_copy(dst_hbm.at[pl.ds(wid * EPW + j * K, K)], idx_s)
        pltpu.sync_copy(buf_v, shared.at[idx_s], add=True)
        return ()
    lax.fori_loop(0, NCH, body, ())
    plsc.subcore_barrier()

    def out_chunk(t, _):
        base = s * RPT + t * RCH
        pltpu.sync_copy(shared.at[pl.ds(base, RCH)], buf_v)
        pltpu.sync_copy(buf_v, deg_out.at[c, pl.ds(base, RCH)])
        return ()
    lax.fori_loop(0, ncopies, out_chunk, ())


@functools.partial(
    pl.kernel,
    out_type=jax.ShapeDtypeStruct((NC, N, H), jnp.float32),
    mesh=_mesh,
    scratch_types=[
        pltpu.VMEM_SHARED((N, H), jnp.float32),
        pltpu.VMEM((K,), jnp.int32),
        pltpu.VMEM((K,), jnp.int32),
        pltpu.VMEM((K, H), jnp.float32),
        pltpu.SemaphoreType.DMA,
    ],
)
def _sc_aggregate(table_hbm, src_hbm, dst_hbm, agg_out,
                  shared, src_s, dst_s, rows_v, sem):
    zb_v = rows_v
    c = lax.axis_index("c")
    s = lax.axis_index("s")
    ncopies = jnp.where(s == NS - 1, 5, 8)

    zero16 = jnp.zeros((16,), jnp.float32)

    def fill_zeros(i, _):
        for q in range(H // 16):
            zb_v[i, pl.ds(q * 16, 16)] = zero16
        return ()
    lax.fori_loop(0, RCH, fill_zeros, (), unroll=2)

    def zero_chunk(t, _):
        pltpu.sync_copy(zb_v, shared.at[pl.ds(s * RPT + t * RCH, RCH)])
        return ()
    lax.fori_loop(0, ncopies, zero_chunk, ())
    plsc.subcore_barrier()

    def body(j, _):
        base = s * EPT + j * K
        pltpu.sync_copy(src_hbm.at[pl.ds(c * E + base, K)], src_s)
        pltpu.sync_copy(dst_hbm.at[pl.ds(base, K)], dst_s)
        pltpu.async_copy(table_hbm.at[src_s], rows_v, sem).wait()
        pltpu.sync_copy(rows_v, shared.at[dst_s], add=True)
        return ()
    lax.fori_loop(0, NCHT, body, ())
    plsc.subcore_barrier()

    def out_chunk(t, _):
        base = s * RPT + t * RCH
        pltpu.sync_copy(shared.at[pl.ds(base, RCH)], zb_v)
        pltpu.sync_copy(zb_v, agg_out.at[c, pl.ds(base, RCH)])
        return ()
    lax.fori_loop(0, ncopies, out_chunk, ())


RB = 1000
NB = N // RB


def _norm_from_deg(deg_blk):
    d = deg_blk[0][:, 0:1] + deg_blk[1][:, 0:1]
    return lax.rsqrt(jnp.maximum(d, 1.0))


def _tc_embed_body(data_ref, we_ref, w1_ref, deg_ref, out_ref):
    x0 = jnp.dot(data_ref[...], we_ref[...], preferred_element_type=jnp.float32)
    hw = jnp.dot(x0, w1_ref[...], preferred_element_type=jnp.float32)
    hwn = hw * _norm_from_deg(deg_ref[...])
    out_ref[0] = hwn[:, :H]
    out_ref[1] = hwn[:, H:]


def _tc_layer2_body(agg_ref, deg_ref, w2_ref, out_ref):
    norm = _norm_from_deg(deg_ref[...])
    a = agg_ref[...]
    x1 = jnp.maximum(jnp.concatenate([a[0], a[1]], axis=1) * norm, 0.0)
    hw = jnp.dot(x1, w2_ref[...], preferred_element_type=jnp.float32)
    hwn = hw * norm
    out_ref[0] = hwn[:, :H]
    out_ref[1] = hwn[:, H:]


def _tc_readout_body(agg_ref, deg_ref, cent_ref, wout_ref, bout_ref,
                     out_ref, acc_ref):
    b = pl.program_id(0)

    @pl.when(b == 0)
    def _init():
        acc_ref[...] = jnp.zeros_like(acc_ref)

    norm = _norm_from_deg(deg_ref[...])
    a = agg_ref[...]
    x2 = jnp.maximum(jnp.concatenate([a[0], a[1]], axis=1) * norm, 0.0)
    cent = cent_ref[...]
    x2s = jnp.sum(x2 * x2, axis=1, keepdims=True)
    c2s = jnp.sum(cent * cent, axis=1)[None, :]
    prod = lax.dot_general(x2, cent, (((1,), (1,)), ((), ())),
                           preferred_element_type=jnp.float32)
    d2 = jnp.maximum(x2s + c2s - 2.0 * prod, 0.0)
    dist = jnp.sqrt(d2 + 1e-8)
    acc_ref[...] += jnp.sum(dist.reshape(RB // 8, 8, 128), axis=0)

    @pl.when(b == NB - 1)
    def _final():
        g = jnp.sum(acc_ref[...], axis=0, keepdims=True) * (1.0 / N)
        logits = jnp.dot(g, wout_ref[...],
                         preferred_element_type=jnp.float32)
        out_ref[...] = jnp.broadcast_to(logits, (8, 128)) + bout_ref[...]


def kernel(data, edge_index, W_embed, W_conv1, W_conv2, centroids, W_out, b_out):
    src = edge_index[0]
    dst = edge_index[1]
    src2 = jnp.concatenate([src, src + N])

    cent_pad = jnp.zeros((128, D), jnp.float32).at[:C].set(centroids)
    wout_pad = jnp.zeros((128, 128), jnp.float32).at[:C, :NCLS].set(W_out)
    bout_pad = jnp.zeros((8, 128), jnp.float32).at[:, :NCLS].set(b_out[None, :])

    deg_p = _sc_degree(dst)

    deg_spec = pl.BlockSpec((NC, RB, H), lambda b: (0, b, 0))
    half_spec = pl.BlockSpec((NC, RB, H), lambda b: (0, b, 0))

    hwn1 = pl.pallas_call(
        _tc_embed_body,
        grid=(NB,),
        in_specs=[
            pl.BlockSpec((RB, DIN), lambda b: (b, 0)),
            pl.BlockSpec((DIN, D), lambda b: (0, 0)),
            pl.BlockSpec((D, D), lambda b: (0, 0)),
            deg_spec,
        ],
        out_specs=half_spec,
        out_shape=jax.ShapeDtypeStruct((NC, N, H), jnp.float32),
    )(data, W_embed, W_conv1, deg_p)

    agg1 = _sc_aggregate(hwn1.reshape(NC * N, H), src2, dst)

    hwn2 = pl.pallas_call(
        _tc_layer2_body,
        grid=(NB,),
        in_specs=[
            half_spec,
            deg_spec,
            pl.BlockSpec((D, D), lambda b: (0, 0)),
        ],
        out_specs=half_spec,
        out_shape=jax.ShapeDtypeStruct((NC, N, H), jnp.float32),
    )(agg1, deg_p, W_conv2)

    agg2 = _sc_aggregate(hwn2.reshape(NC * N, H), src2, dst)

    out_pad = pl.pallas_call(
        _tc_readout_body,
        grid=(NB,),
        in_specs=[
            half_spec,
            deg_spec,
            pl.BlockSpec((128, D), lambda b: (0, 0)),
            pl.BlockSpec((128, 128), lambda b: (0, 0)),
            pl.BlockSpec((8, 128), lambda b: (0, 0)),
        ],
        out_specs=pl.BlockSpec((8, 128), lambda b: (0, 0)),
        out_shape=jax.ShapeDtypeStruct((8, 128), jnp.float32),
        scratch_shapes=[pltpu.VMEM((8, 128), jnp.float32)],
    )(agg2, deg_p, cent_pad, wout_pad, bout_pad)

    return out_pad[0, :NCLS]


# speedup vs baseline: 11.9879x; 2.2068x over previous
import functools

import jax
import jax.numpy as jnp
from jax import lax
from jax.experimental import pallas as pl
from jax.experimental.pallas import tpu as pltpu
from jax.experimental.pallas import tpu_sc as plsc

N = 10000
E = 320000
DIN = 128
D = 256
H = 128
C = 100
NCLS = 10

NC = 2
NS = 16
NW = NC * NS
EPW = E // NW
K = 80
NCH = EPW // K
EPT = E // NS
NCHT = EPT // K
RPT = 640
RCH = 80

_mesh = plsc.VectorSubcoreMesh(
    core_axis_name="c", subcore_axis_name="s", num_cores=NC, num_subcores=NS)


@functools.partial(
    pl.kernel,
    out_type=jax.ShapeDtypeStruct((NC, N, H), jnp.float32),
    mesh=_mesh,
    scratch_types=[
        pltpu.VMEM_SHARED((N, H), jnp.float32),
        pltpu.VMEM((K,), jnp.int32),
        pltpu.VMEM((K, H), jnp.float32),
    ],
)
def _sc_degree(dst_hbm, deg_out, shared, idx_s, buf_v):
    c = lax.axis_index("c")
    s = lax.axis_index("s")
    wid = c * NS + s
    ncopies = jnp.where(s == NS - 1, 5, 8)

    def fill(val):
        v16 = jnp.full((16,), val, jnp.float32)

        def fbody(i, _):
            for q in range(H // 16):
                buf_v[i, pl.ds(q * 16, 16)] = v16
            return ()
        lax.fori_loop(0, K, fbody, (), unroll=2)

    fill(0.0)

    def zero_chunk(t, _):
        pltpu.sync_copy(buf_v, shared.at[pl.ds(s * RPT + t * RCH, RCH)])
        return ()
    lax.fori_loop(0, ncopies, zero_chunk, ())
    plsc.subcore_barrier()

    fill(1.0)

    def body(j, _):
        pltpu.sync_copy(dst_hbm.at[pl.ds(wid * EPW + j * K, K)], idx_s)
        pltpu.sync_copy(buf_v, shared.at[idx_s], add=True)
        return ()
    lax.fori_loop(0, NCH, body, ())
    plsc.subcore_barrier()

    def out_chunk(t, _):
        base = s * RPT + t * RCH
        pltpu.sync_copy(shared.at[pl.ds(base, RCH)], buf_v)
        pltpu.sync_copy(buf_v, deg_out.at[c, pl.ds(base, RCH)])
        return ()
    lax.fori_loop(0, ncopies, out_chunk, ())


G = 10
NGRP = NCHT // G


@functools.partial(
    pl.kernel,
    out_type=jax.ShapeDtypeStruct((NC, N, H), jnp.float32),
    mesh=_mesh,
    scratch_types=[
        pltpu.VMEM_SHARED((N, H), jnp.float32),
        pltpu.VMEM((EPT,), jnp.int32),
        pltpu.VMEM((G, K), jnp.int32),
        pltpu.VMEM((K, H), jnp.float32),
        pltpu.VMEM((K, H), jnp.float32),
        pltpu.SemaphoreType.DMA,
        pltpu.SemaphoreType.DMA,
        pltpu.SemaphoreType.DMA,
        pltpu.SemaphoreType.DMA,
    ],
)
def _sc_aggregate(table_hbm, src_hbm, dst_hbm, agg_out,
                  shared, srcb, dstb, rows0, rows1, g0, g1, s0, s1):
    rows = (rows0, rows1)
    gsem = (g0, g1)
    ssem = (s0, s1)
    zb_v = rows0
    c = lax.axis_index("c")
    s = lax.axis_index("s")
    ncopies = jnp.where(s == NS - 1, 5, 8)

    zero16 = jnp.zeros((16,), jnp.float32)

    def fill_zeros(i, _):
        for q in range(H // 16):
            zb_v[i, pl.ds(q * 16, 16)] = zero16
        return ()
    lax.fori_loop(0, RCH, fill_zeros, (), unroll=2)

    def zero_chunk(t, _):
        pltpu.sync_copy(zb_v, shared.at[pl.ds(s * RPT + t * RCH, RCH)])
        return ()
    lax.fori_loop(0, ncopies, zero_chunk, ())

    pltpu.sync_copy(src_hbm.at[c, s], srcb)
    plsc.subcore_barrier()

    def gath(j, b):
        return pltpu.make_async_copy(
            table_hbm.at[srcb.at[pl.ds(j * K, K)]], rows[b], gsem[b])

    def scat(jj, b):
        return pltpu.make_async_copy(rows[b], shared.at[dstb.at[jj]], ssem[b])

    for b in range(2):
        gath(b, b).start()

    def group_(g, _):
        pltpu.sync_copy(dst_hbm.at[s, g], dstb)

        def round_(r, _):
            for b in range(2):
                jj = r * 2 + b
                j = g * G + jj
                gath(j, b).wait()
                scat(jj, b).start(add=True)

                @pl.when(j < NCHT - 2)
                def _next():
                    scat(jj, b).wait()
                    gath(j + 2, b).start()
            return ()
        lax.fori_loop(0, G // 2, round_, ())
        return ()
    lax.fori_loop(0, NGRP, group_, ())
    for b in range(2):
        scat(G - 2 + b, b).wait()
    plsc.subcore_barrier()

    def out_chunk(t, _):
        base = s * RPT + t * RCH
        pltpu.sync_copy(shared.at[pl.ds(base, RCH)], zb_v)
        pltpu.sync_copy(zb_v, agg_out.at[c, pl.ds(base, RCH)])
        return ()
    lax.fori_loop(0, ncopies, out_chunk, ())


RB = 1000
NB = N // RB


def _norm_from_deg(deg_blk):
    d = deg_blk[0][:, 0:1] + deg_blk[1][:, 0:1]
    return lax.rsqrt(jnp.maximum(d, 1.0))


def _tc_embed_body(data_ref, we_ref, w1_ref, deg_ref, out_ref):
    x0 = jnp.dot(data_ref[...], we_ref[...], preferred_element_type=jnp.float32)
    hw = jnp.dot(x0, w1_ref[...], preferred_element_type=jnp.float32)
    hwn = hw * _norm_from_deg(deg_ref[...])
    out_ref[0] = hwn[:, :H]
    out_ref[1] = hwn[:, H:]


def _tc_layer2_body(agg_ref, deg_ref, w2_ref, out_ref):
    norm = _norm_from_deg(deg_ref[...])
    a = agg_ref[...]
    x1 = jnp.maximum(jnp.concatenate([a[0], a[1]], axis=1) * norm, 0.0)
    hw = jnp.dot(x1, w2_ref[...], preferred_element_type=jnp.float32)
    hwn = hw * norm
    out_ref[0] = hwn[:, :H]
    out_ref[1] = hwn[:, H:]


def _tc_readout_body(agg_ref, deg_ref, cent_ref, wout_ref, bout_ref,
                     out_ref, acc_ref):
    b = pl.program_id(0)

    @pl.when(b == 0)
    def _init():
        acc_ref[...] = jnp.zeros_like(acc_ref)

    norm = _norm_from_deg(deg_ref[...])
    a = agg_ref[...]
    x2 = jnp.maximum(jnp.concatenate([a[0], a[1]], axis=1) * norm, 0.0)
    cent = cent_ref[...]
    x2s = jnp.sum(x2 * x2, axis=1, keepdims=True)
    c2s = jnp.sum(cent * cent, axis=1)[None, :]
    prod = lax.dot_general(x2, cent, (((1,), (1,)), ((), ())),
                           preferred_element_type=jnp.float32)
    d2 = jnp.maximum(x2s + c2s - 2.0 * prod, 0.0)
    dist = jnp.sqrt(d2 + 1e-8)
    acc_ref[...] += jnp.sum(dist.reshape(RB // 8, 8, 128), axis=0)

    @pl.when(b == NB - 1)
    def _final():
        g = jnp.sum(acc_ref[...], axis=0, keepdims=True) * (1.0 / N)
        logits = jnp.dot(g, wout_ref[...],
                         preferred_element_type=jnp.float32)
        out_ref[...] = jnp.broadcast_to(logits, (8, 128)) + bout_ref[...]


def kernel(data, edge_index, W_embed, W_conv1, W_conv2, centroids, W_out, b_out):
    src = edge_index[0]
    dst = edge_index[1]
    src2 = jnp.concatenate([src, src + N]).reshape(NC, NS, EPT)
    dst3 = dst.reshape(NS, NGRP, G, K)

    cent_pad = jnp.zeros((128, D), jnp.float32).at[:C].set(centroids)
    wout_pad = jnp.zeros((128, 128), jnp.float32).at[:C, :NCLS].set(W_out)
    bout_pad = jnp.zeros((8, 128), jnp.float32).at[:, :NCLS].set(b_out[None, :])

    deg_p = _sc_degree(dst)

    deg_spec = pl.BlockSpec((NC, RB, H), lambda b: (0, b, 0))
    half_spec = pl.BlockSpec((NC, RB, H), lambda b: (0, b, 0))

    hwn1 = pl.pallas_call(
        _tc_embed_body,
        grid=(NB,),
        in_specs=[
            pl.BlockSpec((RB, DIN), lambda b: (b, 0)),
            pl.BlockSpec((DIN, D), lambda b: (0, 0)),
            pl.BlockSpec((D, D), lambda b: (0, 0)),
            deg_spec,
        ],
        out_specs=half_spec,
        out_shape=jax.ShapeDtypeStruct((NC, N, H), jnp.float32),
    )(data, W_embed, W_conv1, deg_p)

    agg1 = _sc_aggregate(hwn1.reshape(NC * N, H), src2, dst3)

    hwn2 = pl.pallas_call(
        _tc_layer2_body,
        grid=(NB,),
        in_specs=[
            half_spec,
            deg_spec,
            pl.BlockSpec((D, D), lambda b: (0, 0)),
        ],
        out_specs=half_spec,
        out_shape=jax.ShapeDtypeStruct((NC, N, H), jnp.float32),
    )(agg1, deg_p, W_conv2)

    agg2 = _sc_aggregate(hwn2.reshape(NC * N, H), src2, dst3)

    out_pad = pl.pallas_call(
        _tc_readout_body,
        grid=(NB,),
        in_specs=[
            half_spec,
            deg_spec,
            pl.BlockSpec((128, D), lambda b: (0, 0)),
            pl.BlockSpec((128, 128), lambda b: (0, 0)),
            pl.BlockSpec((8, 128), lambda b: (0, 0)),
        ],
        out_specs=pl.BlockSpec((8, 128), lambda b: (0, 0)),
        out_shape=jax.ShapeDtypeStruct((8, 128), jnp.float32),
        scratch_shapes=[pltpu.VMEM((8, 128), jnp.float32)],
    )(agg2, deg_p, cent_pad, wout_pad, bout_pad)

    return out_pad[0, :NCLS]
